# baseline (speedup 1.0000x reference)
import functools

import jax
import jax.numpy as jnp
from jax.experimental import pallas as pl
from jax.experimental.pallas import tpu as pltpu

_B = 262144
_D = 128
_K = 10
_BBA = 32768
_NBA = _B // _BBA
_BBB = 8192
_NBB = _B // _BBB


def _kernel(z_ref, c_ref, qt_out_ref, pt_out_ref, qt_ref, acc_ref):
    t = pl.program_id(0)

    @pl.when(t < _NBA)
    def _phase_a():
        off = pl.multiple_of(t * _BBA, _BBA)
        zb = z_ref[...]
        cc = c_ref[...]
        zc = jax.lax.dot_general(
            cc, zb, (((1,), (1,)), ((), ())),
            preferred_element_type=jnp.float32)
        zsq = zb * zb
        ones = jnp.ones((1, _D), dtype=jnp.float32)
        zn = jax.lax.dot_general(
            ones, zsq, (((1,), (1,)), ((), ())),
            preferred_element_type=jnp.float32)
        cn = jnp.sum(cc * cc, axis=1, keepdims=True)
        sq = zn + (cn - 2.0 * zc)
        qu = 1.0 / (1.0 + sq)
        rs = jnp.sum(qu, axis=0, keepdims=True)
        qn = qu * (1.0 / rs)
        qt_ref[:, pl.ds(off, _BBA)] = qn
        qt_out_ref[...] = qn
        part = qn[:, 0:128]
        for k in range(1, _BBA // 128):
            part = part + qn[:, k * 128:(k + 1) * 128]

        @pl.when(t == 0)
        def _():
            acc_ref[...] = part

        @pl.when(t > 0)
        def _():
            acc_ref[...] = acc_ref[...] + part

    @pl.when(t >= _NBA)
    def _phase_b():
        j = t - _NBA
        off = pl.multiple_of(j * _BBB, _BBB)
        qt = qt_ref[:, pl.ds(off, _BBB)]
        s = jnp.sum(acc_ref[...], axis=1, keepdims=True)
        w = (qt * qt) * (1.0 / s)
        rs = jnp.sum(w, axis=0, keepdims=True)
        pt = w * (1.0 / rs)
        pt_out_ref[...] = pt


@functools.partial(jax.jit, static_argnames=("interpret",))
def kernel(z, cluster_centers, interpret=False):
    qt, pt = pl.pallas_call(
        _kernel,
        grid=(_NBA + _NBB,),
        in_specs=[
            pl.BlockSpec((_BBA, _D), lambda t: (jnp.minimum(t, _NBA - 1), 0)),
            pl.BlockSpec((_K, _D), lambda t: (0, 0)),
        ],
        out_specs=[
            pl.BlockSpec((_K, _BBA), lambda t: (0, jnp.minimum(t, _NBA - 1))),
            pl.BlockSpec((_K, _BBB),
                         lambda t: (0, jnp.maximum(t - _NBA, 0))),
        ],
        out_shape=[
            jax.ShapeDtypeStruct((_K, _B), jnp.float32),
            jax.ShapeDtypeStruct((_K, _B), jnp.float32),
        ],
        scratch_shapes=[
            pltpu.VMEM((_K, _B), jnp.float32),
            pltpu.VMEM((_K, 128), jnp.float32),
        ],
        compiler_params=pltpu.CompilerParams(
            dimension_semantics=("arbitrary",),
            vmem_limit_bytes=56 * 1024 * 1024,
        ),
        interpret=interpret,
    )(z, cluster_centers)
    return (qt.T, pt.T)


# speedup vs baseline: 1.0860x; 1.0860x over previous
import functools

import jax
import jax.numpy as jnp
from jax.experimental import pallas as pl
from jax.experimental.pallas import tpu as pltpu

_B = 262144
_D = 128
_K = 10
_BBA = 16384
_NBA = _B // _BBA
_BBB = 32768
_NBB = _B // _BBB


def _kernel(z_ref, c_ref, qt_out_ref, pt_out_ref, qt_ref, acc_ref):
    t = pl.program_id(0)

    @pl.when(t < _NBA)
    def _phase_a():
        off = pl.multiple_of(t * _BBA, _BBA)
        zb = z_ref[...]
        cc = c_ref[...]
        zc = jax.lax.dot_general(
            cc, zb, (((1,), (1,)), ((), ())),
            preferred_element_type=jnp.float32)
        zsq = zb * zb
        ones = jnp.ones((1, _D), dtype=jnp.float32)
        zn = jax.lax.dot_general(
            ones, zsq, (((1,), (1,)), ((), ())),
            preferred_element_type=jnp.float32)
        cn = jnp.sum(cc * cc, axis=1, keepdims=True)
        sq = zn + (cn - 2.0 * zc)
        qu = 1.0 / (1.0 + sq)
        rs = jnp.sum(qu, axis=0, keepdims=True)
        qn = qu * (1.0 / rs)
        qt_ref[:, pl.ds(off, _BBA)] = qn * qn
        qt_out_ref[...] = qn
        part = qn[:, 0:128]
        for k in range(1, _BBA // 128):
            part = part + qn[:, k * 128:(k + 1) * 128]

        @pl.when(t == 0)
        def _():
            acc_ref[...] = part

        @pl.when(t > 0)
        def _():
            acc_ref[...] = acc_ref[...] + part

    @pl.when(t >= _NBA)
    def _phase_b():
        j = t - _NBA
        off = pl.multiple_of(j * _BBB, _BBB)
        qt = qt_ref[:, pl.ds(off, _BBB)]
        s = jnp.sum(acc_ref[...], axis=1, keepdims=True)
        w = qt * (1.0 / s)
        rs = jnp.sum(w, axis=0, keepdims=True)
        pt = w * (1.0 / rs)
        pt_out_ref[...] = pt


@functools.partial(jax.jit, static_argnames=("interpret",))
def kernel(z, cluster_centers, interpret=False):
    qt, pt = pl.pallas_call(
        _kernel,
        grid=(_NBA + _NBB,),
        in_specs=[
            pl.BlockSpec((_BBA, _D), lambda t: (jnp.minimum(t, _NBA - 1), 0)),
            pl.BlockSpec((_K, _D), lambda t: (0, 0)),
        ],
        out_specs=[
            pl.BlockSpec((_K, _BBA), lambda t: (0, jnp.minimum(t, _NBA - 1))),
            pl.BlockSpec((_K, _BBB),
                         lambda t: (0, jnp.maximum(t - _NBA, 0))),
        ],
        out_shape=[
            jax.ShapeDtypeStruct((_K, _B), jnp.float32),
            jax.ShapeDtypeStruct((_K, _B), jnp.float32),
        ],
        scratch_shapes=[
            pltpu.VMEM((_K, _B), jnp.float32),
            pltpu.VMEM((_K, 128), jnp.float32),
        ],
        compiler_params=pltpu.CompilerParams(
            dimension_semantics=("arbitrary",),
            vmem_limit_bytes=56 * 1024 * 1024,
        ),
        interpret=interpret,
    )(z, cluster_centers)
    return (qt.T, pt.T)


# speedup vs baseline: 1.0997x; 1.0126x over previous
import functools

import jax
import jax.numpy as jnp
from jax.experimental import pallas as pl
from jax.experimental.pallas import tpu as pltpu

_B = 262144
_D = 128
_K = 10
_BBA = 16384
_NBA = _B // _BBA
_BBB = 65536
_NBB = _B // _BBB


def _kernel(z_ref, c_ref, qt_out_ref, pt_out_ref, qt_ref, acc_ref):
    t = pl.program_id(0)

    @pl.when(t < _NBA)
    def _phase_a():
        off = pl.multiple_of(t * _BBA, _BBA)
        zb = z_ref[...]
        cc = c_ref[...]
        zc = jax.lax.dot_general(
            cc, zb, (((1,), (1,)), ((), ())),
            preferred_element_type=jnp.float32)
        zsq = zb * zb
        ones = jnp.ones((1, _D), dtype=jnp.float32)
        zn = jax.lax.dot_general(
            ones, zsq, (((1,), (1,)), ((), ())),
            preferred_element_type=jnp.float32)
        cn = jnp.sum(cc * cc, axis=1, keepdims=True)
        sq = zn + (cn - 2.0 * zc)
        qu = 1.0 / (1.0 + sq)
        rs = jnp.sum(qu, axis=0, keepdims=True)
        qn = qu * (1.0 / rs)
        qt_ref[:, pl.ds(off, _BBA)] = qn * qn
        qt_out_ref[...] = qn
        part = qn[:, 0:128]
        for k in range(1, _BBA // 128):
            part = part + qn[:, k * 128:(k + 1) * 128]

        @pl.when(t == 0)
        def _():
            acc_ref[...] = part

        @pl.when(t > 0)
        def _():
            acc_ref[...] = acc_ref[...] + part

    @pl.when(t >= _NBA)
    def _phase_b():
        j = t - _NBA
        off = pl.multiple_of(j * _BBB, _BBB)
        qt = qt_ref[:, pl.ds(off, _BBB)]
        s = jnp.sum(acc_ref[...], axis=1, keepdims=True)
        w = qt * (1.0 / s)
        rs = jnp.sum(w, axis=0, keepdims=True)
        pt = w * (1.0 / rs)
        pt_out_ref[...] = pt


@functools.partial(jax.jit, static_argnames=("interpret",))
def kernel(z, cluster_centers, interpret=False):
    qt, pt = pl.pallas_call(
        _kernel,
        grid=(_NBA + _NBB,),
        in_specs=[
            pl.BlockSpec((_BBA, _D), lambda t: (jnp.minimum(t, _NBA - 1), 0)),
            pl.BlockSpec((_K, _D), lambda t: (0, 0)),
        ],
        out_specs=[
            pl.BlockSpec((_K, _BBA), lambda t: (0, jnp.minimum(t, _NBA - 1))),
            pl.BlockSpec((_K, _BBB),
                         lambda t: (0, jnp.maximum(t - _NBA, 0))),
        ],
        out_shape=[
            jax.ShapeDtypeStruct((_K, _B), jnp.float32),
            jax.ShapeDtypeStruct((_K, _B), jnp.float32),
        ],
        scratch_shapes=[
            pltpu.VMEM((_K, _B), jnp.float32),
            pltpu.VMEM((_K, 128), jnp.float32),
        ],
        compiler_params=pltpu.CompilerParams(
            dimension_semantics=("arbitrary",),
            vmem_limit_bytes=56 * 1024 * 1024,
        ),
        interpret=interpret,
    )(z, cluster_centers)
    return (qt.T, pt.T)
